# baseline (speedup 1.0000x reference)
import jax
import jax.numpy as jnp
from jax import lax
from jax.experimental import pallas as pl
from jax.experimental.pallas import tpu as pltpu
from jax.experimental.pallas import tpu_sc as plsc

V = 1000000
NF = 32
B = 16384
NC = 2
NS = 16
NW = NC * NS
L = 16
BPW = B // NW
CHUNK = 128
NCHUNK = BPW // CHUNK
QT = V // 128
TAIL0 = QT * 128
TAILW = V - TAIL0
WQ = 16
NBLK = QT // WQ
NU = NBLK * 4
NWAVE = (NU + NW - 1) // NW
VP = 1000064
FLAT = NF * VP


def _conv_body(tab_u, tab_i, out_u, out_i, bufs, rsem, wsem):
    wid = lax.axis_index("s") * NC + lax.axis_index("c")

    def src_of(tab, k):
        u = wid + NW * k
        blk = u >> 2
        t = u & 3
        return tab.at[pl.ds(8 * t, 8), pl.ds(blk * (WQ * 128), WQ * 128)]

    for tab, out in ((tab_u, out_u), (tab_i, out_i)):
        pltpu.async_copy(src_of(tab, 0), bufs.at[0], rsem)

        def wave(k2, carry, tab=tab, out=out):
            for half in (0, 1):
                k = 2 * k2 + half

                @pl.when(wid + NW * k < NU)
                def _(k=k, half=half, tab=tab, out=out):
                    @pl.when(wid + NW * (k + 1) < NU)
                    def _(k=k, half=half, tab=tab):
                        pltpu.async_copy(src_of(tab, k + 1),
                                         bufs.at[1 - half], rsem)

                    pltpu.make_async_copy(src_of(tab, k), bufs.at[half],
                                          rsem).wait()
                    u = wid + NW * k
                    blk = u >> 2
                    t = u & 3
                    ws = []
                    for s in range(8):
                        dst = out.at[pl.ds((8 * t + s) * VP
                                           + blk * (WQ * 128), WQ * 128)]
                        ws.append(pltpu.async_copy(bufs.at[half, s], dst,
                                                   wsem))
                    for cp in ws:
                        cp.wait()
            return carry

        lax.fori_loop(0, (NWAVE + 1) // 2, wave, 0)

        @pl.when(wid < 8)
        def _(tab=tab, out=out):
            t = wid & 3
            half = (wid >> 2) * 2
            c0 = NBLK * (WQ * 128) + half * 128
            src = tab.at[pl.ds(8 * t, 8), pl.ds(c0, 256)]
            pltpu.async_copy(src, bufs.at[0, :, pl.ds(0, 256)], rsem).wait()
            ws = []
            for s in range(8):
                dst = out.at[pl.ds((8 * t + s) * VP + c0, 256)]
                ws.append(pltpu.async_copy(bufs.at[0, s, pl.ds(0, 256)],
                                           dst, wsem))
            for cp in ws:
                cp.wait()


def _dot_body(conv_u, conv_i, tail_u, tail_i, user, item_i, item_j,
              out_i, out_j, idx_u, idx_i, idx_j, gix_u, gix_i, gix_j,
              vals_u, vals_i, vals_j, tv_u, tv_i, res_i, res_j, sem):
    wid = lax.axis_index("s") * NC + lax.axis_index("c")
    base = wid * BPW

    pltpu.sync_copy(tail_u, tv_u)
    pltpu.sync_copy(tail_i, tv_i)
    for c in range(NCHUNK):
        off = base + c * CHUNK
        pltpu.sync_copy(user.at[pl.ds(off, CHUNK)], idx_u.at[c])
        pltpu.sync_copy(item_i.at[pl.ds(off, CHUNK)], idx_i.at[c])
        pltpu.sync_copy(item_j.at[pl.ds(off, CHUNK)], idx_j.at[c])

    def mkgix(jf, carry):
        o = jf * VP
        for c in range(NCHUNK):
            for t in range(CHUNK // L):
                s = pl.ds(t * L, L)
                gix_u[jf, c, s] = idx_u[c, s] + o
                gix_i[jf, c, s] = idx_i[c, s] + o
                gix_j[jf, c, s] = idx_j[c, s] + o
        return carry

    lax.fori_loop(0, NF, mkgix, 0)

    def gather(jf, carry):
        cps = []
        for c in range(NCHUNK):
            dst = pl.ds(c * CHUNK, CHUNK)
            cps.append(pltpu.async_copy(
                conv_u.at[gix_u.at[jf, c]], vals_u.at[jf, dst], sem))
            cps.append(pltpu.async_copy(
                conv_i.at[gix_i.at[jf, c]], vals_i.at[jf, dst], sem))
            cps.append(pltpu.async_copy(
                conv_i.at[gix_j.at[jf, c]], vals_j.at[jf, dst], sem))
        for cp in cps:
            cp.wait()
        return carry

    lax.fori_loop(0, NF, gather, 0)

    def block(blk, carry):
        s = pl.ds(blk * L, L)
        c = blk >> 3
        o = (blk & 7) * L
        uv = idx_u[c, pl.ds(o, L)]
        iv = idx_i[c, pl.ds(o, L)]
        jv = idx_j[c, pl.ds(o, L)]
        mu = uv >= TAIL0
        mi = iv >= TAIL0
        mj = jv >= TAIL0
        ut = jnp.maximum(uv - TAIL0, 0)
        it = jnp.maximum(iv - TAIL0, 0)
        jt = jnp.maximum(jv - TAIL0, 0)
        acc_i = jnp.zeros((L,), jnp.float32)
        acc_j = jnp.zeros((L,), jnp.float32)
        for jf in range(NF):
            cs = jnp.full((L,), jf, jnp.int32)
            eu = jnp.where(mu, plsc.load_gather(tv_u, [ut, cs]),
                           vals_u[jf, s])
            ei = jnp.where(mi, plsc.load_gather(tv_i, [it, cs]),
                           vals_i[jf, s])
            ej = jnp.where(mj, plsc.load_gather(tv_i, [jt, cs]),
                           vals_j[jf, s])
            acc_i = acc_i + eu * ei
            acc_j = acc_j + eu * ej
        res_i[s] = acc_i
        res_j[s] = acc_j
        return carry

    lax.fori_loop(0, BPW // L, block, 0)

    pltpu.sync_copy(res_i, out_i.at[pl.ds(base, BPW)])
    pltpu.sync_copy(res_j, out_j.at[pl.ds(base, BPW)])


@jax.jit
def _mf_bpr(user, item_i, item_j, embed_user, embed_item):
    mesh = plsc.VectorSubcoreMesh(core_axis_name="c", subcore_axis_name="s")
    conv = pl.kernel(
        _conv_body,
        out_type=(jax.ShapeDtypeStruct((FLAT,), jnp.float32),
                  jax.ShapeDtypeStruct((FLAT,), jnp.float32)),
        mesh=mesh,
        compiler_params=pltpu.CompilerParams(needs_layout_passes=False,
                                             use_tc_tiling_on_sc=True),
        scratch_types=[
            pltpu.VMEM((2, 8, WQ * 128), jnp.float32),
            pltpu.SemaphoreType.DMA,
            pltpu.SemaphoreType.DMA,
        ],
    )
    conv_u, conv_i = conv(embed_user.T, embed_item.T)

    dot = pl.kernel(
        _dot_body,
        out_type=(jax.ShapeDtypeStruct((B,), jnp.float32),
                  jax.ShapeDtypeStruct((B,), jnp.float32)),
        mesh=mesh,
        compiler_params=pltpu.CompilerParams(needs_layout_passes=False,
                                             use_tc_tiling_on_sc=False),
        scratch_types=[
            pltpu.VMEM((NCHUNK, CHUNK), jnp.int32),
            pltpu.VMEM((NCHUNK, CHUNK), jnp.int32),
            pltpu.VMEM((NCHUNK, CHUNK), jnp.int32),
            pltpu.VMEM((NF, NCHUNK, CHUNK), jnp.int32),
            pltpu.VMEM((NF, NCHUNK, CHUNK), jnp.int32),
            pltpu.VMEM((NF, NCHUNK, CHUNK), jnp.int32),
            pltpu.VMEM((NF, BPW), jnp.float32),
            pltpu.VMEM((NF, BPW), jnp.float32),
            pltpu.VMEM((NF, BPW), jnp.float32),
            pltpu.VMEM((TAILW, NF), jnp.float32),
            pltpu.VMEM((TAILW, NF), jnp.float32),
            pltpu.VMEM((BPW,), jnp.float32),
            pltpu.VMEM((BPW,), jnp.float32),
            pltpu.SemaphoreType.DMA,
        ],
    )
    tail_u = embed_user[TAIL0:, :]
    tail_i = embed_item[TAIL0:, :]
    return dot(conv_u, conv_i, tail_u, tail_i, user, item_i, item_j)


def kernel(user, item_i, item_j, embed_user, embed_item):
    return _mf_bpr(user.astype(jnp.int32), item_i.astype(jnp.int32),
                   item_j.astype(jnp.int32), embed_user, embed_item)
